# baseline (speedup 1.0000x reference)
import functools

import jax
import jax.numpy as jnp
from jax import lax
from jax.experimental import pallas as pl
from jax.experimental.pallas import tpu as pltpu
from jax.experimental.pallas import tpu_sc as plsc

N = 10000
D = 128
G = 64
ALPHA = 0.5
K_PROP = 3
NUM_LAYERS = 3

NC, NS = 2, 16
NW = NC * NS
CHUNK = 128
N_PAD = 10112
NBLK = N_PAD // 128
RPT = N_PAD // NS


def _sc_mesh():
    return plsc.VectorSubcoreMesh(
        core_axis_name="c", subcore_axis_name="s", num_cores=NC, num_subcores=NS
    )


def _deg_sc(cols3, ones_w, zeros_nd):
    nch = cols3.shape[1]

    @functools.partial(
        pl.kernel,
        out_type=jax.ShapeDtypeStruct((NC, N_PAD, D), jnp.float32),
        mesh=_sc_mesh(),
        scratch_types=[
            pltpu.VMEM((nch, CHUNK), jnp.int32),
            pltpu.VMEM((CHUNK, D), jnp.float32),
            pltpu.VMEM_SHARED((N_PAD, D), jnp.float32),
        ],
    )
    def k(cols_hbm, ones_hbm, zeros_hbm, out_hbm, cbuf, onesbuf, acc):
        c = lax.axis_index("c")
        s = lax.axis_index("s")
        wid = c * NS + s
        r0 = s * RPT
        pltpu.sync_copy(zeros_hbm.at[pl.ds(r0, RPT)], acc.at[pl.ds(r0, RPT)])
        pltpu.sync_copy(ones_hbm, onesbuf)
        pltpu.sync_copy(cols_hbm.at[wid], cbuf)
        plsc.subcore_barrier()

        def body(j, carry):
            pltpu.sync_copy(onesbuf, acc.at[cbuf.at[j]], add=True)
            return carry

        lax.fori_loop(0, nch, body, 0)
        plsc.subcore_barrier()
        pltpu.sync_copy(acc.at[pl.ds(r0, RPT)], out_hbm.at[c, pl.ds(r0, RPT)])

    return k(cols3, ones_w, zeros_nd)


NBUF = 2
NPASS = 2


def _scatter_sc(y, rows3, cols3, zeros_nd):
    nch = rows3.shape[1]
    npp = nch // NPASS
    assert nch % (NPASS * NBUF) == 0

    @functools.partial(
        pl.kernel,
        out_type=jax.ShapeDtypeStruct((NC, N_PAD, D), jnp.float32),
        mesh=_sc_mesh(),
        scratch_types=(
            [pltpu.VMEM((npp, CHUNK), jnp.int32)] * 2
            + [pltpu.VMEM((CHUNK, D), jnp.float32)] * NBUF
            + [pltpu.VMEM_SHARED((N_PAD, D), jnp.float32)]
            + [pltpu.SemaphoreType.DMA] * NBUF
        ),
    )
    def k(y_hbm, rows_hbm, cols_hbm, zeros_hbm, z_hbm, rbuf, cbuf, *rest):
        gbuf = rest[:NBUF]
        acc = rest[NBUF]
        gsem = rest[NBUF + 1:NBUF + 1 + NBUF]
        c = lax.axis_index("c")
        s = lax.axis_index("s")
        wid = c * NS + s
        r0 = s * RPT

        def start_gather(j, b):
            pltpu.async_copy(y_hbm.at[rbuf.at[j]], gbuf[b], gsem[b])

        def wait_gather(b):
            pltpu.make_async_copy(y_hbm.at[rbuf.at[0]], gbuf[b], gsem[b]).wait()

        def body(j, carry):
            start_gather(j + 1, 1)
            wait_gather(0)
            pltpu.sync_copy(gbuf[0], acc.at[cbuf.at[j]], add=True)
            start_gather(j + 2, 0)
            wait_gather(1)
            pltpu.sync_copy(gbuf[1], acc.at[cbuf.at[j + 1]], add=True)
            return carry

        for pss in range(NPASS):
            pltpu.sync_copy(rows_hbm.at[wid, pl.ds(pss * npp, npp)], rbuf)
            pltpu.sync_copy(cols_hbm.at[wid, pl.ds(pss * npp, npp)], cbuf)
            start_gather(0, 0)
            if pss == 0:
                pltpu.sync_copy(zeros_hbm.at[pl.ds(r0, RPT)],
                                acc.at[pl.ds(r0, RPT)])
                plsc.subcore_barrier()

            lax.fori_loop(0, (npp - 2) // 2, lambda i, cr: body(i * 2, cr), 0)
            start_gather(npp - 1, 1)
            wait_gather(0)
            pltpu.sync_copy(gbuf[0], acc.at[cbuf.at[npp - 2]], add=True)
            wait_gather(1)
            pltpu.sync_copy(gbuf[1], acc.at[cbuf.at[npp - 1]], add=True)

        plsc.subcore_barrier()
        pltpu.sync_copy(acc.at[pl.ds(r0, RPT)], z_hbm.at[c, pl.ds(r0, RPT)])

    return k(y, rows3, cols3, zeros_nd)


_DOT = dict(preferred_element_type=jnp.float32, precision=lax.Precision.HIGHEST)


def _prologue_tc(x_pad, W, b2d, degp):
    def body(x_ref, w_ref, b_ref, deg_ref, h_ref, y_ref, d_ref):
        x1 = lax.dot_general(x_ref[...], w_ref[...], (((1,), (1,)), ((), ())),
                             **_DOT) + b_ref[...]
        degs = deg_ref[0, :, 0:1] + deg_ref[1, :, 0:1] + 1.0
        dinv = lax.rsqrt(degs)
        h_ref[...] = x1
        y_ref[...] = dinv * x1
        d_ref[...] = jnp.broadcast_to(dinv, (128, 8))

    return pl.pallas_call(
        body,
        grid=(NBLK,),
        in_specs=[
            pl.BlockSpec((128, D), lambda i: (i, 0)),
            pl.BlockSpec((D, D), lambda i: (0, 0)),
            pl.BlockSpec((1, D), lambda i: (0, 0)),
            pl.BlockSpec((NC, 128, D), lambda i: (0, i, 0)),
        ],
        out_specs=[
            pl.BlockSpec((128, D), lambda i: (i, 0)),
            pl.BlockSpec((128, D), lambda i: (i, 0)),
            pl.BlockSpec((128, 8), lambda i: (i, 0)),
        ],
        out_shape=[
            jax.ShapeDtypeStruct((N_PAD, D), jnp.float32),
            jax.ShapeDtypeStruct((N_PAD, D), jnp.float32),
            jax.ShapeDtypeStruct((N_PAD, 8), jnp.float32),
        ],
    )(x_pad, W, b2d, degp)


def _combine_tc(z, y, h, dinv8, end_of_layer):
    def body(z_ref, y_ref, h_ref, d_ref, *outs):
        zz = z_ref[0] + z_ref[1]
        dv = d_ref[:, 0:1]
        xn = (1.0 - ALPHA) * (dv * (zz + y_ref[...])) + ALPHA * h_ref[...]
        if end_of_layer:
            xl = jnp.where(xn >= 0, xn, 0.01 * xn)
            outs[0][...] = xl
            outs[1][...] = dv * xl
        else:
            outs[0][...] = dv * xn

    n_out = 2 if end_of_layer else 1
    return pl.pallas_call(
        body,
        grid=(NBLK,),
        in_specs=[
            pl.BlockSpec((NC, 128, D), lambda i: (0, i, 0)),
            pl.BlockSpec((128, D), lambda i: (i, 0)),
            pl.BlockSpec((128, D), lambda i: (i, 0)),
            pl.BlockSpec((128, 8), lambda i: (i, 0)),
        ],
        out_specs=[pl.BlockSpec((128, D), lambda i: (i, 0))] * n_out,
        out_shape=[jax.ShapeDtypeStruct((N_PAD, D), jnp.float32)] * n_out,
    )(z, y, h, dinv8)


def _final_tc(z, y, h, dinv8, batch8):
    def body(z_ref, y_ref, h_ref, d_ref, b_ref, o_ref, sacc, cacc):
        i = pl.program_id(0)

        @pl.when(i == 0)
        def _init():
            sacc[...] = jnp.zeros_like(sacc)
            cacc[...] = jnp.zeros_like(cacc)

        zz = z_ref[0] + z_ref[1]
        dv = d_ref[:, 0:1]
        xn = (1.0 - ALPHA) * (dv * (zz + y_ref[...])) + ALPHA * h_ref[...]
        xl = jnp.where(xn >= 0, xn, 0.01 * xn)
        gids = lax.broadcasted_iota(jnp.int32, (128, G), 1)
        oh = (b_ref[:, 0:1] == gids).astype(jnp.float32)
        sacc[...] += lax.dot_general(oh, xl, (((0,), (0,)), ((), ())), **_DOT)
        cacc[...] += lax.dot_general(oh, jnp.ones((128, D), jnp.float32),
                                     (((0,), (0,)), ((), ())), **_DOT)

        @pl.when(i == NBLK - 1)
        def _fin():
            o_ref[...] = sacc[...] / jnp.maximum(cacc[...], 1.0)

    return pl.pallas_call(
        body,
        grid=(NBLK,),
        in_specs=[
            pl.BlockSpec((NC, 128, D), lambda i: (0, i, 0)),
            pl.BlockSpec((128, D), lambda i: (i, 0)),
            pl.BlockSpec((128, D), lambda i: (i, 0)),
            pl.BlockSpec((128, 8), lambda i: (i, 0)),
            pl.BlockSpec((128, 8), lambda i: (i, 0)),
        ],
        out_specs=pl.BlockSpec((G, D), lambda i: (0, 0)),
        out_shape=jax.ShapeDtypeStruct((G, D), jnp.float32),
        scratch_shapes=[
            pltpu.VMEM((G, D), jnp.float32),
            pltpu.VMEM((G, D), jnp.float32),
        ],
    )(z, y, h, dinv8, batch8)


def kernel(x, edge_index, batch, W_emb, b_emb):
    n = x.shape[0]
    e = edge_index.shape[1]
    nch = -(-e // (NW * CHUNK))
    nch = -(-nch // (NPASS * NBUF)) * (NPASS * NBUF)
    e_pad = NW * nch * CHUNK

    row = edge_index[0]
    col = edge_index[1]
    rowp = jnp.concatenate([row, jnp.zeros((e_pad - e,), jnp.int32)])
    colp = jnp.concatenate([col, jnp.full((e_pad - e,), n, jnp.int32)])
    rows3 = rowp.reshape(NW, nch, CHUNK)
    cols3 = colp.reshape(NW, nch, CHUNK)

    x_pad = jnp.pad(x, ((0, N_PAD - n), (0, 0)))
    batch_pad = jnp.concatenate(
        [batch, jnp.full((N_PAD - n,), G, jnp.int32)])
    batch8 = jnp.broadcast_to(batch_pad[:, None], (N_PAD, 8))
    b2d = b_emb[None, :]

    ones_w = jnp.ones((CHUNK, D), jnp.float32)
    zeros_nd = jnp.zeros((N_PAD, D), jnp.float32)

    degp = _deg_sc(cols3, ones_w, zeros_nd)
    h, y, dinv8 = _prologue_tc(x_pad, W_emb, b2d, degp)

    out = None
    for layer in range(NUM_LAYERS):
        for k in range(K_PROP):
            z = _scatter_sc(y, rows3, cols3, zeros_nd)
            if layer == NUM_LAYERS - 1 and k == K_PROP - 1:
                out = _final_tc(z, y, h, dinv8, batch8)
            elif k == K_PROP - 1:
                h, y = _combine_tc(z, y, h, dinv8, end_of_layer=True)
            else:
                (y,) = _combine_tc(z, y, h, dinv8, end_of_layer=False)
    return out


# speedup vs baseline: 1.4792x; 1.4792x over previous
import functools

import jax
import jax.numpy as jnp
from jax import lax
from jax.experimental import pallas as pl
from jax.experimental.pallas import tpu as pltpu
from jax.experimental.pallas import tpu_sc as plsc

N = 10000
D = 128
G = 64
ALPHA = 0.5
K_PROP = 3
NUM_LAYERS = 3

NC, NS = 2, 16
NW = NC * NS
CHUNK = 128
N_PAD = 10112
NBLK = N_PAD // 128
RBLK = 632
NRB = N_PAD // RBLK
RPT = N_PAD // NS


def _sc_mesh():
    return plsc.VectorSubcoreMesh(
        core_axis_name="c", subcore_axis_name="s", num_cores=NC, num_subcores=NS
    )


def _deg_sc(cols3, ones_w, zeros_nd):
    nch = cols3.shape[1]

    @functools.partial(
        pl.kernel,
        out_type=jax.ShapeDtypeStruct((NC, N_PAD, D), jnp.float32),
        mesh=_sc_mesh(),
        scratch_types=[
            pltpu.VMEM((nch, CHUNK), jnp.int32),
            pltpu.VMEM((CHUNK, D), jnp.float32),
            pltpu.VMEM_SHARED((N_PAD, D), jnp.float32),
        ],
    )
    def k(cols_hbm, ones_hbm, zeros_hbm, out_hbm, cbuf, onesbuf, acc):
        c = lax.axis_index("c")
        s = lax.axis_index("s")
        wid = c * NS + s
        r0 = s * RPT
        pltpu.sync_copy(zeros_hbm.at[pl.ds(r0, RPT)], acc.at[pl.ds(r0, RPT)])
        pltpu.sync_copy(ones_hbm, onesbuf)
        pltpu.sync_copy(cols_hbm.at[wid], cbuf)
        plsc.subcore_barrier()

        def body(j, carry):
            pltpu.sync_copy(onesbuf, acc.at[cbuf.at[j]], add=True)
            return carry

        lax.fori_loop(0, nch, body, 0)
        plsc.subcore_barrier()
        pltpu.sync_copy(acc.at[pl.ds(r0, RPT)], out_hbm.at[c, pl.ds(r0, RPT)])

    return k(cols3, ones_w, zeros_nd)


NBUF = 1


def _scatter_sc(y, rows3, cols3, zeros_nd):
    nch = rows3.shape[1]
    npp = nch

    @functools.partial(
        pl.kernel,
        out_type=jax.ShapeDtypeStruct((NC, N_PAD, D), jnp.float32),
        mesh=_sc_mesh(),
        scratch_types=(
            [pltpu.VMEM((nch, CHUNK), jnp.int32)] * 2
            + [pltpu.VMEM((CHUNK, D), jnp.float32)] * NBUF
            + [pltpu.VMEM_SHARED((N_PAD, D), jnp.float32)]
            + [pltpu.SemaphoreType.DMA] * NBUF
        ),
    )
    def k(y_hbm, rows_hbm, cols_hbm, zeros_hbm, z_hbm, rbuf, cbuf, *rest):
        gbuf = rest[:NBUF]
        acc = rest[NBUF]
        gsem = rest[NBUF + 1:NBUF + 1 + NBUF]
        c = lax.axis_index("c")
        s = lax.axis_index("s")
        wid = c * NS + s
        r0 = s * RPT

        def start_gather(j, b):
            pltpu.async_copy(y_hbm.at[rbuf.at[j]], gbuf[b], gsem[b])

        def wait_gather(b):
            pltpu.make_async_copy(y_hbm.at[rbuf.at[0]], gbuf[b], gsem[b]).wait()

        pltpu.sync_copy(zeros_hbm.at[pl.ds(r0, RPT)], acc.at[pl.ds(r0, RPT)])
        pltpu.sync_copy(rows_hbm.at[wid], rbuf)
        pltpu.sync_copy(cols_hbm.at[wid], cbuf)
        plsc.subcore_barrier()

        def body(j, carry):
            pltpu.async_copy(y_hbm.at[rbuf.at[j]], gbuf[0], gsem[0]).wait()
            pltpu.sync_copy(gbuf[0], acc.at[cbuf.at[j]], add=True)
            return carry

        lax.fori_loop(0, nch, body, 0)

        plsc.subcore_barrier()
        pltpu.sync_copy(acc.at[pl.ds(r0, RPT)], z_hbm.at[c, pl.ds(r0, RPT)])

    return k(y, rows3, cols3, zeros_nd)


_DOT = dict(preferred_element_type=jnp.float32, precision=lax.Precision.HIGHEST)


def _prologue_tc(x_pad, W, b2d, degp):
    def body(x_ref, w_ref, b_ref, deg_ref, h_ref, y_ref, d_ref):
        x1 = lax.dot_general(x_ref[...], w_ref[...], (((1,), (1,)), ((), ())),
                             **_DOT) + b_ref[...]
        degs = deg_ref[0, :, 0:1] + deg_ref[1, :, 0:1] + 1.0
        dinv = lax.rsqrt(degs)
        h_ref[...] = x1
        y_ref[...] = dinv * x1
        d_ref[...] = jnp.broadcast_to(dinv, (RBLK, 8))

    return pl.pallas_call(
        body,
        grid=(NRB,),
        in_specs=[
            pl.BlockSpec((RBLK, D), lambda i: (i, 0)),
            pl.BlockSpec((D, D), lambda i: (0, 0)),
            pl.BlockSpec((1, D), lambda i: (0, 0)),
            pl.BlockSpec((NC, RBLK, D), lambda i: (0, i, 0)),
        ],
        out_specs=[
            pl.BlockSpec((RBLK, D), lambda i: (i, 0)),
            pl.BlockSpec((RBLK, D), lambda i: (i, 0)),
            pl.BlockSpec((RBLK, 8), lambda i: (i, 0)),
        ],
        out_shape=[
            jax.ShapeDtypeStruct((N_PAD, D), jnp.float32),
            jax.ShapeDtypeStruct((N_PAD, D), jnp.float32),
            jax.ShapeDtypeStruct((N_PAD, 8), jnp.float32),
        ],
    )(x_pad, W, b2d, degp)


def _combine_tc(z, y, h, dinv8, end_of_layer):
    def body(z_ref, y_ref, h_ref, d_ref, *outs):
        zz = z_ref[0] + z_ref[1]
        dv = d_ref[:, 0:1]
        xn = (1.0 - ALPHA) * (dv * (zz + y_ref[...])) + ALPHA * h_ref[...]
        if end_of_layer:
            xl = jnp.where(xn >= 0, xn, 0.01 * xn)
            outs[0][...] = xl
            outs[1][...] = dv * xl
        else:
            outs[0][...] = dv * xn

    n_out = 2 if end_of_layer else 1
    return pl.pallas_call(
        body,
        grid=(NRB,),
        in_specs=[
            pl.BlockSpec((NC, RBLK, D), lambda i: (0, i, 0)),
            pl.BlockSpec((RBLK, D), lambda i: (i, 0)),
            pl.BlockSpec((RBLK, D), lambda i: (i, 0)),
            pl.BlockSpec((RBLK, 8), lambda i: (i, 0)),
        ],
        out_specs=[pl.BlockSpec((RBLK, D), lambda i: (i, 0))] * n_out,
        out_shape=[jax.ShapeDtypeStruct((N_PAD, D), jnp.float32)] * n_out,
    )(z, y, h, dinv8)


def _final_tc(z, y, h, dinv8, batch8):
    def body(z_ref, y_ref, h_ref, d_ref, b_ref, o_ref, sacc, cacc):
        i = pl.program_id(0)

        @pl.when(i == 0)
        def _init():
            sacc[...] = jnp.zeros_like(sacc)
            cacc[...] = jnp.zeros_like(cacc)

        zz = z_ref[0] + z_ref[1]
        dv = d_ref[:, 0:1]
        xn = (1.0 - ALPHA) * (dv * (zz + y_ref[...])) + ALPHA * h_ref[...]
        xl = jnp.where(xn >= 0, xn, 0.01 * xn)
        gids = lax.broadcasted_iota(jnp.int32, (RBLK, G), 1)
        oh = (b_ref[:, 0:1] == gids).astype(jnp.float32)
        sacc[...] += lax.dot_general(oh, xl, (((0,), (0,)), ((), ())), **_DOT)
        cacc[...] += lax.dot_general(oh, jnp.ones((RBLK, D), jnp.float32),
                                     (((0,), (0,)), ((), ())), **_DOT)

        @pl.when(i == NRB - 1)
        def _fin():
            o_ref[...] = sacc[...] / jnp.maximum(cacc[...], 1.0)

    return pl.pallas_call(
        body,
        grid=(NRB,),
        in_specs=[
            pl.BlockSpec((NC, RBLK, D), lambda i: (0, i, 0)),
            pl.BlockSpec((RBLK, D), lambda i: (i, 0)),
            pl.BlockSpec((RBLK, D), lambda i: (i, 0)),
            pl.BlockSpec((RBLK, 8), lambda i: (i, 0)),
            pl.BlockSpec((RBLK, 8), lambda i: (i, 0)),
        ],
        out_specs=pl.BlockSpec((G, D), lambda i: (0, 0)),
        out_shape=jax.ShapeDtypeStruct((G, D), jnp.float32),
        scratch_shapes=[
            pltpu.VMEM((G, D), jnp.float32),
            pltpu.VMEM((G, D), jnp.float32),
        ],
    )(z, y, h, dinv8, batch8)


def kernel(x, edge_index, batch, W_emb, b_emb):
    n = x.shape[0]
    e = edge_index.shape[1]
    nch = -(-e // (NW * CHUNK))
    e_pad = NW * nch * CHUNK

    row = edge_index[0]
    col = edge_index[1]
    rowp = jnp.concatenate([row, jnp.zeros((e_pad - e,), jnp.int32)])
    colp = jnp.concatenate([col, jnp.full((e_pad - e,), n, jnp.int32)])
    rows3 = rowp.reshape(NW, nch, CHUNK)
    cols3 = colp.reshape(NW, nch, CHUNK)

    x_pad = jnp.pad(x, ((0, N_PAD - n), (0, 0)))
    batch_pad = jnp.concatenate(
        [batch, jnp.full((N_PAD - n,), G, jnp.int32)])
    batch8 = jnp.broadcast_to(batch_pad[:, None], (N_PAD, 8))
    b2d = b_emb[None, :]

    ones_w = jnp.ones((CHUNK, D), jnp.float32)
    zeros_nd = jnp.zeros((N_PAD, D), jnp.float32)

    degp = _deg_sc(cols3, ones_w, zeros_nd)
    h, y, dinv8 = _prologue_tc(x_pad, W_emb, b2d, degp)

    out = None
    for layer in range(NUM_LAYERS):
        for k in range(K_PROP):
            z = _scatter_sc(y, rows3, cols3, zeros_nd)
            if layer == NUM_LAYERS - 1 and k == K_PROP - 1:
                out = _final_tc(z, y, h, dinv8, batch8)
            elif k == K_PROP - 1:
                h, y = _combine_tc(z, y, h, dinv8, end_of_layer=True)
            else:
                (y,) = _combine_tc(z, y, h, dinv8, end_of_layer=False)
    return out
